# baseline (speedup 1.0000x reference)
import functools

import jax
import jax.numpy as jnp
from jax.experimental import pallas as pl
from jax.experimental.pallas import tpu as pltpu

_S = 392
_INV_S = 1.0 / _S


def _rowmean_mxu_kernel(x_ref, o_ref, *, tr):
    x = x_ref[...]
    ones = jnp.ones((1, x.shape[1]), jnp.float32)
    s = jax.lax.dot_general(ones, x, (((1,), (1,)), ((), ())),
                            preferred_element_type=jnp.float32)
    o_ref[...] = s.reshape(tr // 128, 128) * _INV_S


def _rowmean_vpu_kernel(x_ref, o_ref, *, tr):
    x = x_ref[...]
    folded = x[:, 0:128] + x[:, 128:256] + x[:, 256:384]
    total = (jnp.sum(folded, axis=-1, keepdims=True)
             + jnp.sum(x[:, 384:392], axis=-1, keepdims=True)) * _INV_S
    o_ref[...] = total.reshape(tr // 128, 128)


_KERNELS = {"mxu": _rowmean_mxu_kernel, "vpu": _rowmean_vpu_kernel}


def _rowmean_split_kernel(a_ref, b_ref, c_ref, t_ref, o_ref, *, tr, s):
    folded = a_ref[...] + b_ref[...] + c_ref[...]
    total = (jnp.sum(folded, axis=-1, keepdims=True)
             + jnp.sum(t_ref[:, 0:s - 384], axis=-1, keepdims=True)) * _INV_S
    o_ref[...] = total.reshape(tr // 128, 128)


def _rowmean_split(x2d, *, tr):
    rows, s = x2d.shape
    grid = (rows // tr,)
    dense = pl.pallas_call(
        functools.partial(_rowmean_split_kernel, tr=tr, s=s),
        out_shape=jax.ShapeDtypeStruct((rows // 128, 128), x2d.dtype),
        grid=grid,
        in_specs=[
            pl.BlockSpec((tr, 128), lambda i: (i, 0)),
            pl.BlockSpec((tr, 128), lambda i: (i, 1)),
            pl.BlockSpec((tr, 128), lambda i: (i, 2)),
            pl.BlockSpec((tr, 128), lambda i: (i, 3)),
        ],
        out_specs=pl.BlockSpec((tr // 128, 128), lambda i: (i, 0)),
        compiler_params=pltpu.CompilerParams(
            dimension_semantics=("parallel",)),
    )(x2d, x2d, x2d, x2d)
    return dense.reshape(rows, 1)


def _rowmean(x2d, *, tr, body):
    rows, s = x2d.shape
    grid = (rows // tr,)
    dense = pl.pallas_call(
        functools.partial(_KERNELS[body], tr=tr),
        out_shape=jax.ShapeDtypeStruct((rows // 128, 128), x2d.dtype),
        grid=grid,
        in_specs=[pl.BlockSpec((tr, s), lambda i: (i, 0))],
        out_specs=pl.BlockSpec((tr // 128, 128), lambda i: (i, 0)),
        compiler_params=pltpu.CompilerParams(
            dimension_semantics=("parallel",)),
    )(x2d)
    return dense.reshape(rows, 1)


def _rowmean_split8_kernel(a0, b0, c0, t0, a1, b1, c1, t1, o_ref, *, tr, s):
    tail = s - 384
    f0 = a0[...] + b0[...] + c0[...]
    f1 = a1[...] + b1[...] + c1[...]
    tot0 = (jnp.sum(f0, axis=-1, keepdims=True)
            + jnp.sum(t0[:, 0:tail], axis=-1, keepdims=True))
    tot1 = (jnp.sum(f1, axis=-1, keepdims=True)
            + jnp.sum(t1[:, 0:tail], axis=-1, keepdims=True))
    total = jnp.concatenate([tot0, tot1], axis=0) * _INV_S
    o_ref[...] = total.reshape(tr // 128, 128)


def _rowmean_split8(x2d, *, tr):
    rows, s = x2d.shape
    grid = (rows // tr,)
    h = tr // 2
    specs = []
    for p in (0, 1):
        for j in (0, 1, 2, 3):
            specs.append(pl.BlockSpec((h, 128), lambda i, p=p, j=j: (2 * i + p, j)))
    dense = pl.pallas_call(
        functools.partial(_rowmean_split8_kernel, tr=tr, s=s),
        out_shape=jax.ShapeDtypeStruct((rows // 128, 128), x2d.dtype),
        grid=grid,
        in_specs=specs,
        out_specs=pl.BlockSpec((tr // 128, 128), lambda i: (i, 0)),
        compiler_params=pltpu.CompilerParams(
            dimension_semantics=("parallel",)),
    )(*([x2d] * 8))
    return dense.reshape(rows, 1)


def kernel(x2d):
    return _rowmean_split8(x2d, tr=4096)


# speedup vs baseline: 1.0457x; 1.0457x over previous
import functools

import jax
import jax.numpy as jnp
from jax.experimental import pallas as pl
from jax.experimental.pallas import tpu as pltpu

_S = 392
_INV_S = 1.0 / _S


def _rowmean_mxu_kernel(x_ref, o_ref, *, tr):
    x = x_ref[...]
    ones = jnp.ones((1, x.shape[1]), jnp.float32)
    s = jax.lax.dot_general(ones, x, (((1,), (1,)), ((), ())),
                            preferred_element_type=jnp.float32)
    o_ref[...] = s.reshape(tr // 128, 128) * _INV_S


def _rowmean_vpu_kernel(x_ref, o_ref, *, tr):
    x = x_ref[...]
    folded = x[:, 0:128] + x[:, 128:256] + x[:, 256:384]
    total = (jnp.sum(folded, axis=-1, keepdims=True)
             + jnp.sum(x[:, 384:392], axis=-1, keepdims=True)) * _INV_S
    o_ref[...] = total.reshape(tr // 128, 128)


_KERNELS = {"mxu": _rowmean_mxu_kernel, "vpu": _rowmean_vpu_kernel}


def _rowmean_split_kernel(a_ref, b_ref, c_ref, t_ref, o_ref, *, tr, s):
    t = t_ref[...]
    lane = jax.lax.broadcasted_iota(jnp.int32, t.shape, 1)
    folded = (a_ref[...] + b_ref[...] + c_ref[...]
              + jnp.where(lane < s - 384, t, 0.0))
    total = jnp.sum(folded, axis=-1, keepdims=True) * _INV_S
    o_ref[...] = total.reshape(tr // 128, 128)


def _rowmean_split(x2d, *, tr):
    rows, s = x2d.shape
    grid = (rows // tr,)
    dense = pl.pallas_call(
        functools.partial(_rowmean_split_kernel, tr=tr, s=s),
        out_shape=jax.ShapeDtypeStruct((rows // 128, 128), x2d.dtype),
        grid=grid,
        in_specs=[
            pl.BlockSpec((tr, 128), lambda i: (i, 0)),
            pl.BlockSpec((tr, 128), lambda i: (i, 1)),
            pl.BlockSpec((tr, 128), lambda i: (i, 2)),
            pl.BlockSpec((tr, 128), lambda i: (i, 3)),
        ],
        out_specs=pl.BlockSpec((tr // 128, 128), lambda i: (i, 0)),
        compiler_params=pltpu.CompilerParams(
            dimension_semantics=("parallel",)),
    )(x2d, x2d, x2d, x2d)
    return dense.reshape(rows, 1)


def _rowmean(x2d, *, tr, body):
    rows, s = x2d.shape
    grid = (rows // tr,)
    dense = pl.pallas_call(
        functools.partial(_KERNELS[body], tr=tr),
        out_shape=jax.ShapeDtypeStruct((rows // 128, 128), x2d.dtype),
        grid=grid,
        in_specs=[pl.BlockSpec((tr, s), lambda i: (i, 0))],
        out_specs=pl.BlockSpec((tr // 128, 128), lambda i: (i, 0)),
        compiler_params=pltpu.CompilerParams(
            dimension_semantics=("parallel",)),
    )(x2d)
    return dense.reshape(rows, 1)


def _rowmean_split8_kernel(a0, b0, c0, t0, a1, b1, c1, t1, o_ref, *, tr, s):
    tail = s - 384
    f0 = a0[...] + b0[...] + c0[...]
    f1 = a1[...] + b1[...] + c1[...]
    tot0 = (jnp.sum(f0, axis=-1, keepdims=True)
            + jnp.sum(t0[:, 0:tail], axis=-1, keepdims=True))
    tot1 = (jnp.sum(f1, axis=-1, keepdims=True)
            + jnp.sum(t1[:, 0:tail], axis=-1, keepdims=True))
    total = jnp.concatenate([tot0, tot1], axis=0) * _INV_S
    o_ref[...] = total.reshape(tr // 128, 128)


def _rowmean_split8(x2d, *, tr):
    rows, s = x2d.shape
    grid = (rows // tr,)
    h = tr // 2
    specs = []
    for p in (0, 1):
        for j in (0, 1, 2, 3):
            specs.append(pl.BlockSpec((h, 128), lambda i, p=p, j=j: (2 * i + p, j)))
    dense = pl.pallas_call(
        functools.partial(_rowmean_split8_kernel, tr=tr, s=s),
        out_shape=jax.ShapeDtypeStruct((rows // 128, 128), x2d.dtype),
        grid=grid,
        in_specs=specs,
        out_specs=pl.BlockSpec((tr // 128, 128), lambda i: (i, 0)),
        compiler_params=pltpu.CompilerParams(
            dimension_semantics=("parallel",)),
    )(*([x2d] * 8))
    return dense.reshape(rows, 1)


def kernel(x2d):
    return _rowmean_split(x2d, tr=4096)
